# baseline (speedup 1.0000x reference)
import functools

import jax
import jax.numpy as jnp
from jax import lax
from jax.experimental import pallas as pl
from jax.experimental.pallas import tpu as pltpu
from jax.experimental.pallas import tpu_sc as plsc

N = 10000
E = 640000
IN = 3
W1 = 256
W2 = 256
H = 512

NC = 2
NS = 16
NW = NC * NS
K = 80
EPW = E // NW
EPS = E // NS
ZCH = 640

f32 = jnp.float32
i32 = jnp.int32

def _mesh():
    return plsc.VectorSubcoreMesh(core_axis_name="c", subcore_axis_name="s")


def _lazy(builder):
    cache = {}

    def call(*args):
        if "k" not in cache:
            cache["k"] = builder()
        return cache["k"](*args)

    return call


def _chunk_off(s):
    return jnp.where(s == NS - 1, N - ZCH, s * ZCH)


def _deg_body(dst_hbm, zeros_hbm, out_hbm, didxA, didxB, ones, acc,
              semA, semB):
    c = lax.axis_index("c")
    s = lax.axis_index("s")
    for r in range(K):
        for q in range(128 // 16):
            ones[r, pl.ds(q * 16, 16)] = jnp.ones((16,), f32)
    off = _chunk_off(s)
    pltpu.sync_copy(zeros_hbm.at[pl.ds(off, ZCH)], acc.at[pl.ds(off, ZCH)])
    plsc.subcore_barrier()
    base = (c * NS + s) * EPW

    def body(i, carry):
        bA = base + (2 * i) * K
        bB = bA + K
        pltpu.sync_copy(dst_hbm.at[pl.ds(bA, K)], didxA)
        pltpu.sync_copy(dst_hbm.at[pl.ds(bB, K)], didxB)
        pltpu.sync_copy(ones, acc.at[didxA], add=True)
        pltpu.sync_copy(ones, acc.at[didxB], add=True)
        return carry

    lax.fori_loop(0, EPW // (2 * K), body, 0)
    plsc.subcore_barrier()
    pltpu.sync_copy(acc.at[pl.ds(off, ZCH)],
                    out_hbm.at[pl.ds(c * N + off, ZCH)])


_deg_kernel = _lazy(lambda: pl.kernel(
    _deg_body,
    out_type=jax.ShapeDtypeStruct((NC * N, 128), f32),
    mesh=_mesh(),
    scratch_types=[
        pltpu.VMEM((K,), i32),
        pltpu.VMEM((K,), i32),
        pltpu.VMEM((K, 128), f32),
        pltpu.VMEM_SHARED((N, 128), f32),
        pltpu.SemaphoreType.DMA,
        pltpu.SemaphoreType.DMA,
    ],
))


def _make_agg_body(per_sub, split_edges, kk, nb):
    def body_fn(*refs):
        tbl_hbm, src_hbm, dst_hbm, zeros_hbm, out_hbm = refs[:5]
        sidx = refs[5:5 + nb]
        didx = refs[5 + nb:5 + 2 * nb]
        rows = refs[5 + 2 * nb:5 + 3 * nb]
        acc = refs[5 + 3 * nb]
        sems = refs[6 + 3 * nb:6 + 4 * nb]
        c = lax.axis_index("c")
        s = lax.axis_index("s")
        off = _chunk_off(s)
        pltpu.sync_copy(zeros_hbm.at[pl.ds(off, ZCH)], acc.at[pl.ds(off, ZCH)])
        plsc.subcore_barrier()
        if split_edges:
            base = (c * NS + s) * per_sub
        else:
            base = s * per_sub
        coff = (c * N).astype(i32)

        def body(i, carry):
            b0 = base + i * (nb * kk)
            gs = []
            for t in range(nb):
                bt = b0 + t * kk
                pltpu.sync_copy(src_hbm.at[pl.ds(bt, kk)], sidx[t])
                if not split_edges:
                    for j in range(kk // 16):
                        sidx[t][pl.ds(j * 16, 16)] = (
                            sidx[t][pl.ds(j * 16, 16)]
                            + jnp.broadcast_to(coff, (16,))
                        )
                pltpu.sync_copy(dst_hbm.at[pl.ds(bt, kk)], didx[t])
                gs.append(pltpu.async_copy(tbl_hbm.at[sidx[t]], rows[t],
                                           sems[t]))
            for t in range(nb):
                gs[t].wait()
                pltpu.sync_copy(rows[t], acc.at[didx[t]], add=True)
            return carry

        lax.fori_loop(0, per_sub // (nb * kk), body, 0)
        plsc.subcore_barrier()
        pltpu.sync_copy(acc.at[pl.ds(off, ZCH)],
                        out_hbm.at[pl.ds(c * N + off, ZCH)])

    return body_fn


def _agg_scratch(kk, nb):
    return (
        [pltpu.VMEM((kk,), i32) for _ in range(nb)]
        + [pltpu.VMEM((kk,), i32) for _ in range(nb)]
        + [pltpu.VMEM((kk, 128), f32) for _ in range(nb)]
        + [pltpu.VMEM_SHARED((N, 128), f32)]
        + [pltpu.SemaphoreType.DMA for _ in range(nb)]
    )


_agg1_kernel = _lazy(lambda: pl.kernel(
    _make_agg_body(EPW, True, 40, 4),
    out_type=jax.ShapeDtypeStruct((NC * N, 128), f32),
    mesh=_mesh(),
    scratch_types=_agg_scratch(40, 4),
))

_agg2_kernel = _lazy(lambda: pl.kernel(
    _make_agg_body(EPS, False, 80, 4),
    out_type=jax.ShapeDtypeStruct((NC * N, 128), f32),
    mesh=_mesh(),
    scratch_types=_agg_scratch(80, 4),
))


BLK = 2000


def _tc_h1_body(p_ref, dis_ref, w_ref, b_ref, h1_ref, g_ref):
    a = (p_ref[0] + p_ref[1]) * dis_ref[...]
    h1 = jnp.dot(a, w_ref[...], preferred_element_type=f32,
                 precision=lax.Precision.HIGHEST) + b_ref[...]
    h1 = jnp.maximum(h1, 0.0)
    h1_ref[...] = h1
    g = h1 * dis_ref[...]
    g_ref[0] = g[:, :128]
    g_ref[1] = g[:, 128:]


def _tc_h1(p, dis_col, Wg1p, bg1):
    return pl.pallas_call(
        _tc_h1_body,
        grid=(N // BLK,),
        in_specs=[
            pl.BlockSpec((2, BLK, 128), lambda i: (0, i, 0)),
            pl.BlockSpec((BLK, 1), lambda i: (i, 0)),
            pl.BlockSpec((128, W1), lambda i: (0, 0)),
            pl.BlockSpec((1, W1), lambda i: (0, 0)),
        ],
        out_specs=[
            pl.BlockSpec((BLK, W1), lambda i: (i, 0)),
            pl.BlockSpec((2, BLK, 128), lambda i: (0, i, 0)),
        ],
        out_shape=[
            jax.ShapeDtypeStruct((N, W1), f32),
            jax.ShapeDtypeStruct((2, N, 128), f32),
        ],
    )(p, dis_col, Wg1p, bg1)


def _lrelu(v):
    return jnp.where(v >= 0.0, v, 0.01 * v)


def _tc_head_body(q_ref, dis_ref, h1_ref, wg2_ref, bg2_ref, wf1_ref, bf1_ref,
                  wf2_ref, bf2_ref, wf3_ref, bf3_ref, out_ref, colsum):
    i = pl.program_id(0)
    hp = lax.Precision.HIGHEST
    agg2 = jnp.concatenate([q_ref[0], q_ref[1]], axis=1) * dis_ref[...]
    h2 = jnp.dot(agg2, wg2_ref[...], preferred_element_type=f32,
                 precision=hp) + bg2_ref[...]
    h = jnp.concatenate([h1_ref[...], h2], axis=1)
    c1 = _lrelu(jnp.dot(h, wf1_ref[...], preferred_element_type=f32,
                        precision=hp) + bf1_ref[...])
    c2 = _lrelu(jnp.dot(c1, wf2_ref[...], preferred_element_type=f32,
                        precision=hp) + bf2_ref[...])
    part = jnp.sum(c2, axis=0, keepdims=True)

    @pl.when(i == 0)
    def _():
        colsum[...] = part

    @pl.when(i > 0)
    def _():
        colsum[...] = colsum[...] + part

    @pl.when(i == pl.num_programs(0) - 1)
    def _():
        out_ref[...] = (jnp.dot(colsum[...], wf3_ref[...],
                                preferred_element_type=f32, precision=hp)
                        + float(N) * bf3_ref[...])


def _tc_head(q, dis_col, h1, Wg2, bg2, Wf1, bf1, Wf2, bf2, Wf3, bf3):
    return pl.pallas_call(
        _tc_head_body,
        grid=(N // BLK,),
        in_specs=[
            pl.BlockSpec((2, BLK, 128), lambda i: (0, i, 0)),
            pl.BlockSpec((BLK, 1), lambda i: (i, 0)),
            pl.BlockSpec((BLK, W1), lambda i: (i, 0)),
            pl.BlockSpec((W2, W2), lambda i: (0, 0)),
            pl.BlockSpec((1, W2), lambda i: (0, 0)),
            pl.BlockSpec((2 * W2, H), lambda i: (0, 0)),
            pl.BlockSpec((1, H), lambda i: (0, 0)),
            pl.BlockSpec((H, H), lambda i: (0, 0)),
            pl.BlockSpec((1, H), lambda i: (0, 0)),
            pl.BlockSpec((H, 1), lambda i: (0, 0)),
            pl.BlockSpec((1, 1), lambda i: (0, 0)),
        ],
        out_specs=pl.BlockSpec((1, 1), lambda i: (0, 0)),
        out_shape=jax.ShapeDtypeStruct((1, 1), f32),
        scratch_shapes=[pltpu.VMEM((1, H), f32)],
    )(q, dis_col, h1, Wg2, bg2, Wf1, bf1, Wf2, bf2, Wf3, bf3)


def kernel(x, edge_index, Wg1, bg1, Wg2, bg2, Wf1, bf1, Wf2, bf2, Wf3, bf3):
    src = edge_index[0]
    dst = edge_index[1]
    zeros128 = jnp.zeros((N, 128), f32)

    degp = _deg_kernel(dst, zeros128)
    deg = degp[:N, 0] + degp[N:, 0]
    dis = lax.rsqrt(jnp.maximum(deg, 1.0))
    dis_col = dis[:, None]

    xd = jnp.pad(x, ((0, 0), (0, 128 - IN))) * dis_col
    agg1p = _agg1_kernel(xd, src, dst, zeros128)
    p = jnp.stack([agg1p[:N], agg1p[N:]])

    Wg1p = jnp.pad(Wg1, ((0, 128 - IN), (0, 0)))
    h1, g = _tc_h1(p, dis_col, Wg1p, bg1[None, :])

    gcat = g.reshape(2 * N, 128)
    agg2p = _agg2_kernel(gcat, src, dst, zeros128)
    q = agg2p.reshape(2, N, 128)

    value = _tc_head(q, dis_col, h1, Wg2, bg2[None, :],
                     Wf1, bf1[None, :], Wf2, bf2[None, :],
                     Wf3, bf3[None, :])
    return value.reshape(1)


# speedup vs baseline: 1.0414x; 1.0414x over previous
import functools

import jax
import jax.numpy as jnp
from jax import lax
from jax.experimental import pallas as pl
from jax.experimental.pallas import tpu as pltpu
from jax.experimental.pallas import tpu_sc as plsc

N = 10000
E = 640000
IN = 3
W1 = 256
W2 = 256
H = 512

NC = 2
NS = 16
NW = NC * NS
K = 80
EPW = E // NW
EPS = E // NS
ZCH = 640

f32 = jnp.float32
i32 = jnp.int32

def _mesh():
    return plsc.VectorSubcoreMesh(core_axis_name="c", subcore_axis_name="s")


def _lazy(builder):
    cache = {}

    def call(*args):
        if "k" not in cache:
            cache["k"] = builder()
        return cache["k"](*args)

    return call


def _chunk_off(s):
    return jnp.where(s == NS - 1, N - ZCH, s * ZCH)


def _deg_body(dst_hbm, zeros_hbm, out_hbm, didxA, didxB, ones, acc,
              semA, semB):
    c = lax.axis_index("c")
    s = lax.axis_index("s")
    for r in range(K):
        for q in range(128 // 16):
            ones[r, pl.ds(q * 16, 16)] = jnp.ones((16,), f32)
    off = _chunk_off(s)
    pltpu.sync_copy(zeros_hbm.at[pl.ds(off, ZCH)], acc.at[pl.ds(off, ZCH)])
    plsc.subcore_barrier()
    base = (c * NS + s) * EPW

    def body(i, carry):
        bA = base + (2 * i) * K
        bB = bA + K
        pltpu.sync_copy(dst_hbm.at[pl.ds(bA, K)], didxA)
        pltpu.sync_copy(dst_hbm.at[pl.ds(bB, K)], didxB)
        pltpu.sync_copy(ones, acc.at[didxA], add=True)
        pltpu.sync_copy(ones, acc.at[didxB], add=True)
        return carry

    lax.fori_loop(0, EPW // (2 * K), body, 0)
    plsc.subcore_barrier()
    pltpu.sync_copy(acc.at[pl.ds(off, ZCH)],
                    out_hbm.at[pl.ds(c * N + off, ZCH)])


_deg_kernel = _lazy(lambda: pl.kernel(
    _deg_body,
    out_type=jax.ShapeDtypeStruct((NC * N, 128), f32),
    mesh=_mesh(),
    scratch_types=[
        pltpu.VMEM((K,), i32),
        pltpu.VMEM((K,), i32),
        pltpu.VMEM((K, 128), f32),
        pltpu.VMEM_SHARED((N, 128), f32),
        pltpu.SemaphoreType.DMA,
        pltpu.SemaphoreType.DMA,
    ],
))


def _make_agg_body(per_sub, split_edges, kk, nb):
    def body_fn(*refs):
        tbl_hbm, src_hbm, dst_hbm, zeros_hbm, out_hbm = refs[:5]
        sidx = refs[5:5 + nb]
        didx = refs[5 + nb:5 + 2 * nb]
        rows = refs[5 + 2 * nb:5 + 3 * nb]
        acc = refs[5 + 3 * nb]
        sems = refs[6 + 3 * nb:6 + 4 * nb]
        c = lax.axis_index("c")
        s = lax.axis_index("s")
        off = _chunk_off(s)
        pltpu.sync_copy(zeros_hbm.at[pl.ds(off, ZCH)], acc.at[pl.ds(off, ZCH)])
        plsc.subcore_barrier()
        if split_edges:
            base = (c * NS + s) * per_sub
        else:
            base = s * per_sub
        coff = (c * N).astype(i32)

        def body(i, carry):
            b0 = base + i * (nb * kk)
            gs = []
            for t in range(nb):
                bt = b0 + t * kk
                pltpu.sync_copy(src_hbm.at[pl.ds(bt, kk)], sidx[t])
                if not split_edges:
                    for j in range(kk // 16):
                        sidx[t][pl.ds(j * 16, 16)] = (
                            sidx[t][pl.ds(j * 16, 16)]
                            + jnp.broadcast_to(coff, (16,))
                        )
                pltpu.sync_copy(dst_hbm.at[pl.ds(bt, kk)], didx[t])
                gs.append(pltpu.async_copy(tbl_hbm.at[sidx[t]], rows[t],
                                           sems[t]))
            for t in range(nb):
                gs[t].wait()
                pltpu.sync_copy(rows[t], acc.at[didx[t]], add=True)
            return carry

        lax.fori_loop(0, per_sub // (nb * kk), body, 0)
        plsc.subcore_barrier()
        pltpu.sync_copy(acc.at[pl.ds(off, ZCH)],
                        out_hbm.at[pl.ds(c * N + off, ZCH)])

    return body_fn


def _agg_scratch(kk, nb):
    return (
        [pltpu.VMEM((kk,), i32) for _ in range(nb)]
        + [pltpu.VMEM((kk,), i32) for _ in range(nb)]
        + [pltpu.VMEM((kk, 128), f32) for _ in range(nb)]
        + [pltpu.VMEM_SHARED((N, 128), f32)]
        + [pltpu.SemaphoreType.DMA for _ in range(nb)]
    )


_agg1_kernel = _lazy(lambda: pl.kernel(
    _make_agg_body(EPW, True, 40, 4),
    out_type=jax.ShapeDtypeStruct((NC * N, 128), f32),
    mesh=_mesh(),
    scratch_types=_agg_scratch(40, 4),
))

_agg2_kernel = _lazy(lambda: pl.kernel(
    _make_agg_body(EPS, False, 80, 4),
    out_type=jax.ShapeDtypeStruct((NC * N, 128), f32),
    mesh=_mesh(),
    scratch_types=_agg_scratch(80, 4),
))


BLK = 2000


def _tc_h1_body(p_ref, dis_ref, w_ref, b_ref, h1_ref, g_ref):
    a = (p_ref[0] + p_ref[1]) * dis_ref[...]
    h1 = jnp.dot(a, w_ref[...], preferred_element_type=f32) + b_ref[...]
    h1 = jnp.maximum(h1, 0.0)
    h1_ref[...] = h1
    g = h1 * dis_ref[...]
    g_ref[0] = g[:, :128]
    g_ref[1] = g[:, 128:]


def _tc_h1(p, dis_col, Wg1p, bg1):
    return pl.pallas_call(
        _tc_h1_body,
        grid=(N // BLK,),
        in_specs=[
            pl.BlockSpec((2, BLK, 128), lambda i: (0, i, 0)),
            pl.BlockSpec((BLK, 1), lambda i: (i, 0)),
            pl.BlockSpec((128, W1), lambda i: (0, 0)),
            pl.BlockSpec((1, W1), lambda i: (0, 0)),
        ],
        out_specs=[
            pl.BlockSpec((BLK, W1), lambda i: (i, 0)),
            pl.BlockSpec((2, BLK, 128), lambda i: (0, i, 0)),
        ],
        out_shape=[
            jax.ShapeDtypeStruct((N, W1), f32),
            jax.ShapeDtypeStruct((2, N, 128), f32),
        ],
    )(p, dis_col, Wg1p, bg1)


def _lrelu(v):
    return jnp.where(v >= 0.0, v, 0.01 * v)


def _tc_head_body(q_ref, dis_ref, h1_ref, wg2_ref, bg2_ref, wf1_ref, bf1_ref,
                  wf2_ref, bf2_ref, wf3_ref, bf3_ref, out_ref, colsum):
    i = pl.program_id(0)
    agg2 = jnp.concatenate([q_ref[0], q_ref[1]], axis=1) * dis_ref[...]
    h2 = jnp.dot(agg2, wg2_ref[...], preferred_element_type=f32) + bg2_ref[...]
    h = jnp.concatenate([h1_ref[...], h2], axis=1)
    c1 = _lrelu(jnp.dot(h, wf1_ref[...],
                        preferred_element_type=f32) + bf1_ref[...])
    c2 = _lrelu(jnp.dot(c1, wf2_ref[...],
                        preferred_element_type=f32) + bf2_ref[...])
    c3 = jnp.dot(c2, wf3_ref[...], preferred_element_type=f32) + bf3_ref[...]
    part = jnp.sum(c3, axis=0, keepdims=True)

    @pl.when(i == 0)
    def _():
        colsum[...] = part

    @pl.when(i > 0)
    def _():
        colsum[...] = colsum[...] + part

    @pl.when(i == pl.num_programs(0) - 1)
    def _():
        out_ref[...] = colsum[...]


def _tc_head(q, dis_col, h1, Wg2, bg2, Wf1, bf1, Wf2, bf2, Wf3, bf3):
    return pl.pallas_call(
        _tc_head_body,
        grid=(N // BLK,),
        in_specs=[
            pl.BlockSpec((2, BLK, 128), lambda i: (0, i, 0)),
            pl.BlockSpec((BLK, 1), lambda i: (i, 0)),
            pl.BlockSpec((BLK, W1), lambda i: (i, 0)),
            pl.BlockSpec((W2, W2), lambda i: (0, 0)),
            pl.BlockSpec((1, W2), lambda i: (0, 0)),
            pl.BlockSpec((2 * W2, H), lambda i: (0, 0)),
            pl.BlockSpec((1, H), lambda i: (0, 0)),
            pl.BlockSpec((H, H), lambda i: (0, 0)),
            pl.BlockSpec((1, H), lambda i: (0, 0)),
            pl.BlockSpec((H, 1), lambda i: (0, 0)),
            pl.BlockSpec((1, 1), lambda i: (0, 0)),
        ],
        out_specs=pl.BlockSpec((1, 1), lambda i: (0, 0)),
        out_shape=jax.ShapeDtypeStruct((1, 1), f32),
        scratch_shapes=[pltpu.VMEM((1, 1), f32)],
    )(q, dis_col, h1, Wg2, bg2, Wf1, bf1, Wf2, bf2, Wf3, bf3)


def kernel(x, edge_index, Wg1, bg1, Wg2, bg2, Wf1, bf1, Wf2, bf2, Wf3, bf3):
    src = edge_index[0]
    dst = edge_index[1]
    zeros128 = jnp.zeros((N, 128), f32)

    degp = _deg_kernel(dst, zeros128)
    deg = degp[:N, 0] + degp[N:, 0]
    dis = lax.rsqrt(jnp.maximum(deg, 1.0))
    dis_col = dis[:, None]

    xd = jnp.pad(x, ((0, 0), (0, 128 - IN))) * dis_col
    agg1p = _agg1_kernel(xd, src, dst, zeros128)
    p = jnp.stack([agg1p[:N], agg1p[N:]])

    Wg1p = jnp.pad(Wg1, ((0, 128 - IN), (0, 0)))
    h1, g = _tc_h1(p, dis_col, Wg1p, bg1[None, :])

    gcat = g.reshape(2 * N, 128)
    agg2p = _agg2_kernel(gcat, src, dst, zeros128)
    q = agg2p.reshape(2, N, 128)

    value = _tc_head(q, dis_col, h1, Wg2, bg2[None, :],
                     Wf1, bf1[None, :], Wf2, bf2[None, :],
                     Wf3, bf3[None, :])
    return value.reshape(1)


# speedup vs baseline: 1.1987x; 1.1511x over previous
import functools

import jax
import jax.numpy as jnp
from jax import lax
from jax.experimental import pallas as pl
from jax.experimental.pallas import tpu as pltpu
from jax.experimental.pallas import tpu_sc as plsc

N = 10000
E = 640000
IN = 3
W1 = 256
W2 = 256
H = 512

NC = 2
NS = 16
NW = NC * NS
K = 80
EPW = E // NW
EPS = E // NS
ZCH = 640

f32 = jnp.float32
i32 = jnp.int32

def _mesh():
    return plsc.VectorSubcoreMesh(core_axis_name="c", subcore_axis_name="s")


def _lazy(builder):
    cache = {}

    def call(*args):
        if "k" not in cache:
            cache["k"] = builder()
        return cache["k"](*args)

    return call


def _chunk_off(s):
    return jnp.where(s == NS - 1, N - ZCH, s * ZCH)


def _deg_body(dst_hbm, zeros_hbm, out_hbm, didxA, didxB, ones, acc,
              semA, semB):
    c = lax.axis_index("c")
    s = lax.axis_index("s")
    for r in range(K):
        for q in range(128 // 16):
            ones[r, pl.ds(q * 16, 16)] = jnp.ones((16,), f32)
    off = _chunk_off(s)
    pltpu.sync_copy(zeros_hbm.at[pl.ds(off, ZCH)], acc.at[pl.ds(off, ZCH)])
    plsc.subcore_barrier()
    base = (c * NS + s) * EPW

    def body(i, carry):
        bA = base + (2 * i) * K
        bB = bA + K

        @pl.when(i > 0)
        def _():
            pltpu.make_async_copy(ones, acc.at[didxA], semA).wait()

        pltpu.sync_copy(dst_hbm.at[pl.ds(bA, K)], didxA)
        pltpu.async_copy(ones, acc.at[didxA], semA, add=True)

        @pl.when(i > 0)
        def _():
            pltpu.make_async_copy(ones, acc.at[didxB], semB).wait()

        pltpu.sync_copy(dst_hbm.at[pl.ds(bB, K)], didxB)
        pltpu.async_copy(ones, acc.at[didxB], semB, add=True)
        return carry

    lax.fori_loop(0, EPW // (2 * K), body, 0)
    pltpu.make_async_copy(ones, acc.at[didxA], semA).wait()
    pltpu.make_async_copy(ones, acc.at[didxB], semB).wait()
    plsc.subcore_barrier()
    pltpu.sync_copy(acc.at[pl.ds(off, ZCH)],
                    out_hbm.at[pl.ds(c * N + off, ZCH)])


_deg_kernel = _lazy(lambda: pl.kernel(
    _deg_body,
    out_type=jax.ShapeDtypeStruct((NC * N, 128), f32),
    mesh=_mesh(),
    scratch_types=[
        pltpu.VMEM((K,), i32),
        pltpu.VMEM((K,), i32),
        pltpu.VMEM((K, 128), f32),
        pltpu.VMEM_SHARED((N, 128), f32),
        pltpu.SemaphoreType.DMA,
        pltpu.SemaphoreType.DMA,
    ],
))


def _make_agg_body(per_sub, split_edges, kk, nb):
    def body_fn(*refs):
        tbl_hbm, src_hbm, dst_hbm, zeros_hbm, out_hbm = refs[:5]
        sidx = refs[5:5 + nb]
        didx = refs[5 + nb:5 + 2 * nb]
        rows = refs[5 + 2 * nb:5 + 3 * nb]
        acc = refs[5 + 3 * nb]
        sems = refs[6 + 3 * nb:6 + 4 * nb]
        ssems = refs[6 + 4 * nb:6 + 5 * nb]
        c = lax.axis_index("c")
        s = lax.axis_index("s")
        off = _chunk_off(s)
        pltpu.sync_copy(zeros_hbm.at[pl.ds(off, ZCH)], acc.at[pl.ds(off, ZCH)])
        plsc.subcore_barrier()
        if split_edges:
            base = (c * NS + s) * per_sub
        else:
            base = s * per_sub
        coff = (c * N).astype(i32)

        def body(i, carry):
            b0 = base + i * (nb * kk)
            gs = []
            for t in range(nb):
                bt = b0 + t * kk

                @pl.when(i > 0)
                def _(t=t):
                    pltpu.make_async_copy(rows[t], acc.at[didx[t]],
                                          ssems[t]).wait()

                pltpu.sync_copy(src_hbm.at[pl.ds(bt, kk)], sidx[t])
                if not split_edges:
                    for j in range(kk // 16):
                        sidx[t][pl.ds(j * 16, 16)] = (
                            sidx[t][pl.ds(j * 16, 16)]
                            + jnp.broadcast_to(coff, (16,))
                        )
                pltpu.sync_copy(dst_hbm.at[pl.ds(bt, kk)], didx[t])
                gs.append(pltpu.async_copy(tbl_hbm.at[sidx[t]], rows[t],
                                           sems[t]))
            for t in range(nb):
                gs[t].wait()
                pltpu.async_copy(rows[t], acc.at[didx[t]], ssems[t], add=True)
            return carry

        lax.fori_loop(0, per_sub // (nb * kk), body, 0)
        for t in range(nb):
            pltpu.make_async_copy(rows[t], acc.at[didx[t]], ssems[t]).wait()
        plsc.subcore_barrier()
        pltpu.sync_copy(acc.at[pl.ds(off, ZCH)],
                        out_hbm.at[pl.ds(c * N + off, ZCH)])

    return body_fn


def _agg_scratch(kk, nb):
    return (
        [pltpu.VMEM((kk,), i32) for _ in range(nb)]
        + [pltpu.VMEM((kk,), i32) for _ in range(nb)]
        + [pltpu.VMEM((kk, 128), f32) for _ in range(nb)]
        + [pltpu.VMEM_SHARED((N, 128), f32)]
        + [pltpu.SemaphoreType.DMA for _ in range(nb)]
        + [pltpu.SemaphoreType.DMA for _ in range(nb)]
    )


_agg1_kernel = _lazy(lambda: pl.kernel(
    _make_agg_body(EPW, True, 40, 4),
    out_type=jax.ShapeDtypeStruct((NC * N, 128), f32),
    mesh=_mesh(),
    scratch_types=_agg_scratch(40, 4),
))

_agg2_kernel = _lazy(lambda: pl.kernel(
    _make_agg_body(EPS, False, 80, 4),
    out_type=jax.ShapeDtypeStruct((NC * N, 128), f32),
    mesh=_mesh(),
    scratch_types=_agg_scratch(80, 4),
))


BLK = 2000


def _tc_h1_body(p_ref, dis_ref, w_ref, b_ref, h1_ref, g_ref):
    a = (p_ref[0] + p_ref[1]) * dis_ref[...]
    h1 = jnp.dot(a, w_ref[...], preferred_element_type=f32) + b_ref[...]
    h1 = jnp.maximum(h1, 0.0)
    h1_ref[...] = h1
    g = h1 * dis_ref[...]
    g_ref[0] = g[:, :128]
    g_ref[1] = g[:, 128:]


def _tc_h1(p, dis_col, Wg1p, bg1):
    return pl.pallas_call(
        _tc_h1_body,
        grid=(N // BLK,),
        in_specs=[
            pl.BlockSpec((2, BLK, 128), lambda i: (0, i, 0)),
            pl.BlockSpec((BLK, 1), lambda i: (i, 0)),
            pl.BlockSpec((128, W1), lambda i: (0, 0)),
            pl.BlockSpec((1, W1), lambda i: (0, 0)),
        ],
        out_specs=[
            pl.BlockSpec((BLK, W1), lambda i: (i, 0)),
            pl.BlockSpec((2, BLK, 128), lambda i: (0, i, 0)),
        ],
        out_shape=[
            jax.ShapeDtypeStruct((N, W1), f32),
            jax.ShapeDtypeStruct((2, N, 128), f32),
        ],
    )(p, dis_col, Wg1p, bg1)


def _lrelu(v):
    return jnp.where(v >= 0.0, v, 0.01 * v)


def _tc_head_body(q_ref, dis_ref, h1_ref, wg2_ref, bg2_ref, wf1_ref, bf1_ref,
                  wf2_ref, bf2_ref, wf3_ref, bf3_ref, out_ref, colsum):
    i = pl.program_id(0)
    agg2 = jnp.concatenate([q_ref[0], q_ref[1]], axis=1) * dis_ref[...]
    h2 = jnp.dot(agg2, wg2_ref[...], preferred_element_type=f32) + bg2_ref[...]
    h = jnp.concatenate([h1_ref[...], h2], axis=1)
    c1 = _lrelu(jnp.dot(h, wf1_ref[...],
                        preferred_element_type=f32) + bf1_ref[...])
    c2 = _lrelu(jnp.dot(c1, wf2_ref[...],
                        preferred_element_type=f32) + bf2_ref[...])
    c3 = jnp.dot(c2, wf3_ref[...], preferred_element_type=f32) + bf3_ref[...]
    part = jnp.sum(c3, axis=0, keepdims=True)

    @pl.when(i == 0)
    def _():
        colsum[...] = part

    @pl.when(i > 0)
    def _():
        colsum[...] = colsum[...] + part

    @pl.when(i == pl.num_programs(0) - 1)
    def _():
        out_ref[...] = colsum[...]


def _tc_head(q, dis_col, h1, Wg2, bg2, Wf1, bf1, Wf2, bf2, Wf3, bf3):
    return pl.pallas_call(
        _tc_head_body,
        grid=(N // BLK,),
        in_specs=[
            pl.BlockSpec((2, BLK, 128), lambda i: (0, i, 0)),
            pl.BlockSpec((BLK, 1), lambda i: (i, 0)),
            pl.BlockSpec((BLK, W1), lambda i: (i, 0)),
            pl.BlockSpec((W2, W2), lambda i: (0, 0)),
            pl.BlockSpec((1, W2), lambda i: (0, 0)),
            pl.BlockSpec((2 * W2, H), lambda i: (0, 0)),
            pl.BlockSpec((1, H), lambda i: (0, 0)),
            pl.BlockSpec((H, H), lambda i: (0, 0)),
            pl.BlockSpec((1, H), lambda i: (0, 0)),
            pl.BlockSpec((H, 1), lambda i: (0, 0)),
            pl.BlockSpec((1, 1), lambda i: (0, 0)),
        ],
        out_specs=pl.BlockSpec((1, 1), lambda i: (0, 0)),
        out_shape=jax.ShapeDtypeStruct((1, 1), f32),
        scratch_shapes=[pltpu.VMEM((1, 1), f32)],
    )(q, dis_col, h1, Wg2, bg2, Wf1, bf1, Wf2, bf2, Wf3, bf3)


def kernel(x, edge_index, Wg1, bg1, Wg2, bg2, Wf1, bf1, Wf2, bf2, Wf3, bf3):
    src = edge_index[0]
    dst = edge_index[1]
    zeros128 = jnp.zeros((N, 128), f32)

    degp = _deg_kernel(dst, zeros128)
    deg = degp[:N, 0] + degp[N:, 0]
    dis = lax.rsqrt(jnp.maximum(deg, 1.0))
    dis_col = dis[:, None]

    xd = jnp.pad(x, ((0, 0), (0, 128 - IN))) * dis_col
    agg1p = _agg1_kernel(xd, src, dst, zeros128)
    p = jnp.stack([agg1p[:N], agg1p[N:]])

    Wg1p = jnp.pad(Wg1, ((0, 128 - IN), (0, 0)))
    h1, g = _tc_h1(p, dis_col, Wg1p, bg1[None, :])

    gcat = g.reshape(2 * N, 128)
    agg2p = _agg2_kernel(gcat, src, dst, zeros128)
    q = agg2p.reshape(2, N, 128)

    value = _tc_head(q, dis_col, h1, Wg2, bg2[None, :],
                     Wf1, bf1[None, :], Wf2, bf2[None, :],
                     Wf3, bf3[None, :])
    return value.reshape(1)
